# baseline (speedup 1.0000x reference)
import functools

import jax
import jax.numpy as jnp
from jax import lax
from jax.experimental import pallas as pl
from jax.experimental.pallas import tpu as pltpu
from jax.experimental.pallas import tpu_sc as plsc

CROP = 7

NC = 2
NS = 16
L = 16
NW = NC * NS
HALF = L // 2


def _roialign_sc(n, c, w, m, bpt, stage_n, npair_main, npair_last):
    mesh = plsc.VectorSubcoreMesh(
        core_axis_name="c", subcore_axis_name="s", num_cores=NC,
        num_subcores=NS)
    rows_per_half = 2 * CROP * HALF

    @functools.partial(
        pl.kernel,
        out_type=jax.ShapeDtypeStruct((m * c * CROP,), jnp.float32),
        mesh=mesh,
        compiler_params=pltpu.CompilerParams(needs_layout_passes=False),
        scratch_types=[
            pltpu.VMEM((stage_n,), jnp.float32),
            pltpu.VMEM((stage_n,), jnp.float32),
            pltpu.VMEM((stage_n,), jnp.int32),
            pltpu.VMEM((2, rows_per_half), jnp.int32),
            pltpu.VMEM((2, rows_per_half), jnp.int32),
            pltpu.VMEM((L * L,), jnp.float32),
            pltpu.VMEM((L * L,), jnp.float32),
            pltpu.VMEM((rows_per_half, c), jnp.float32),
            pltpu.VMEM((rows_per_half, c), jnp.float32),
            pltpu.VMEM((L * c * CROP,), jnp.float32),
            pltpu.VMEM((L * c * CROP,), jnp.float32),
            pltpu.SemaphoreType.DMA,
            pltpu.SemaphoreType.DMA,
            pltpu.SemaphoreType.DMA,
        ],
    )
    def kern(x1_hbm, x2_hbm, bi_hbm, fmt_hbm, out_hbm,
             x1c, x2c, bic, ix0, ix1, wv0, wv1, rA, rB, tb0, tb1,
             semA, semB, semW):
        wid = lax.axis_index("s") * NC + lax.axis_index("c")
        base = wid * bpt
        pltpu.sync_copy(x1_hbm.at[pl.ds(base, stage_n)], x1c)
        pltpu.sync_copy(x2_hbm.at[pl.ds(base, stage_n)], x2c)
        pltpu.sync_copy(bi_hbm.at[pl.ds(base, stage_n)], bic)

        last_tile = wid == NW - 1
        npair = jnp.where(last_tile, npair_last, npair_main)

        lane = lax.iota(jnp.int32, L)
        lane7 = lane * CROP
        lane16 = lane * L
        half_sel = jnp.where(lane < HALF, 0, 1)
        lane_in_half = lane % HALF
        wm1f = float(w - 1)

        def compute(sg, ix, wv):
            gbase = sg * L
            x1 = x1c[pl.ds(gbase, L)]
            x2 = x2c[pl.ds(gbase, L)]
            rowb = bic[pl.ds(gbase, L)] * w
            sp = (x2 - x1) / float(CROP)
            x1n = (x1 + sp * 0.5 - 0.5) / wm1f
            x2n = x1n + sp * float(CROP - 1) / wm1f
            step = (x2n - x1n) * wm1f / float(CROP - 1)
            xs0 = x1n * wm1f
            for j in range(CROP):
                xs = xs0 + float(j) * step
                x0i = xs.astype(jnp.int32)
                i0 = jnp.clip(x0i, 0, w - 1)
                d0 = lane_in_half + (2 * j) * HALF
                plsc.store_scatter(ix, [half_sel, d0], rowb + i0)
                plsc.store_scatter(ix, [half_sel, d0 + HALF],
                                   rowb + jnp.minimum(i0 + 1, w - 1))
                f = xs - x0i.astype(jnp.float32)
                vf = jnp.where((xs >= 0.0) & (xs <= wm1f), 1.0, 0.0)
                w1 = f * vf
                plsc.store_scatter(wv, [lane16 + j], vf - w1)
                plsc.store_scatter(wv, [lane16 + (8 + j)], w1)

        def fire(ix):
            pltpu.async_copy(fmt_hbm.at[ix.at[0]], rA, semA)
            pltpu.async_copy(fmt_hbm.at[ix.at[1]], rB, semB)

        def wait_g(ix, rbuf, sem):
            pltpu.make_async_copy(fmt_hbm.at[ix.at[0]], rbuf, sem).wait()

        def blend(rbuf, wv, half, tb):
            def box(k, _):
                kk = half * HALF + k
                wk = wv[pl.ds(kk * L, L)]
                tbase = lane7 + kk * (c * CROP)
                for j in range(CROP):
                    a0 = wk[j]
                    a1 = wk[8 + j]
                    r0 = (2 * j) * HALF + k
                    r1 = r0 + HALF
                    for cc in range(c // L):
                        g0 = rbuf[r0, pl.ds(cc * L, L)]
                        g1 = rbuf[r1, pl.ds(cc * L, L)]
                        plsc.store_scatter(
                            tb, [tbase + (cc * L * CROP + j)],
                            g0 * a0 + g1 * a1)
                return 0

            lax.fori_loop(0, HALF, box, 0)

        def fire_w(tb, sg):
            pltpu.async_copy(
                tb, out_hbm.at[pl.ds((base + sg * L) * (c * CROP),
                                     L * c * CROP)], semW)

        def wait_w(tb):
            pltpu.make_async_copy(
                tb, out_hbm.at[pl.ds(0, L * c * CROP)], semW).wait()

        compute(0, ix0, wv0)
        fire(ix0)
        wait_g(ix0, rA, semA)
        blend(rA, wv0, 0, tb0)
        wait_g(ix0, rB, semB)
        blend(rB, wv0, 1, tb0)
        compute(1, ix1, wv1)
        fire(ix1)
        fire_w(tb0, 0)

        def pair(p, _):
            s = 1 + 2 * p
            nsg = 2 * npair + 1
            wait_g(ix1, rA, semA)
            blend(rA, wv1, 0, tb1)
            wait_g(ix1, rB, semB)
            blend(rB, wv1, 1, tb1)
            compute(s + 1, ix0, wv0)
            fire(ix0)
            wait_w(tb0)
            fire_w(tb1, s)
            wait_g(ix0, rA, semA)
            blend(rA, wv0, 0, tb0)
            wait_g(ix0, rB, semB)
            blend(rB, wv0, 1, tb0)

            @pl.when(s + 2 < nsg)
            def _():
                compute(s + 2, ix1, wv1)
                fire(ix1)

            wait_w(tb1)
            fire_w(tb0, s + 1)
            return 0

        lax.fori_loop(0, npair, pair, 0)
        wait_w(tb0)

    return kern


def kernel(featuremap, boxes, box_ind):
    n, c, w = featuremap.shape
    m = boxes.shape[0]
    assert m % L == 0 and c % L == 0
    bpt = (m // (NW * L)) * L
    last = m - (NW - 1) * bpt
    assert last % L == 0 and 0 < last
    stage_n = max(bpt, last)
    nsg_main, nsg_last = bpt // L, last // L
    assert nsg_main % 2 == 1 and nsg_last % 2 == 1

    fm_t = jnp.transpose(featuremap, (0, 2, 1)).reshape(n * w, c)
    out = _roialign_sc(n, c, w, m, bpt, stage_n,
                       (nsg_main - 1) // 2, (nsg_last - 1) // 2)(
        boxes[:, 0], boxes[:, 1], box_ind, fm_t)
    return out.reshape(m, c, CROP)
